# baseline (speedup 1.0000x reference)
import functools
import jax
import jax.numpy as jnp
from jax import lax
from jax.experimental import pallas as pl
from jax.experimental.pallas import tpu as pltpu
from jax.experimental.pallas import tpu_sc as plsc

NQ = 8192
NV = 6890
NVP = 6912
NW = 32
QPW = NQ // NW
NGRP = QPW // 16
CHUNK = 96
NCH = NVP // CHUNK
BIG = 3.0e38


def _d2_kernel(xt_ref, vp_ref, o_ref):
    xt = xt_ref[...]
    vp = vp_ref[...]
    xv = jnp.dot(vp.astype(jnp.bfloat16), xt.astype(jnp.bfloat16),
                 preferred_element_type=jnp.float32)
    x2 = (xt[0:1, :] * xt[0:1, :] + xt[1:2, :] * xt[1:2, :]) \
        + xt[2:3, :] * xt[2:3, :]
    v2 = (vp[:, 0:1] * vp[:, 0:1] + vp[:, 1:2] * vp[:, 1:2]) \
        + vp[:, 2:3] * vp[:, 2:3]
    t = x2 + v2
    o_ref[...] = jnp.maximum(t - 2.0 * xv, 1e-12)


def _tc_d2(xqt, vp3):
    BM, BN = 384, 2048
    return pl.pallas_call(
        _d2_kernel,
        out_shape=jax.ShapeDtypeStruct((NVP, NQ), jnp.float32),
        grid=(NVP // BM, NQ // BN),
        in_specs=[
            pl.BlockSpec((3, BN), lambda i, j: (0, j)),
            pl.BlockSpec((BM, 3), lambda i, j: (i, 0)),
        ],
        out_specs=pl.BlockSpec((BM, BN), lambda i, j: (i, j)),
    )(xqt, vp3)


def _splat_i32(x):
    return jnp.zeros((16,), jnp.int32) + x


def _take(v, idx):
    return lax.gather(
        v, idx[:, None],
        lax.GatherDimensionNumbers(offset_dims=(), collapsed_slice_dims=(0,),
                                   start_index_map=(0,)),
        slice_sizes=(1,), mode=lax.GatherScatterMode.PROMISE_IN_BOUNDS)


def _bcast_lane(v, q):
    return _take(v, _splat_i32(q))


def _allsum(v):
    lanes = lax.iota(jnp.int32, 16)
    for s in (8, 4, 2, 1):
        v = v + _take(v, (lanes + s) & 15)
    return v


def _newton_sqrt(x):
    i = lax.bitcast_convert_type(x, jnp.int32)
    g = lax.bitcast_convert_type((i >> 1) + jnp.int32(0x1FBD1DF5), jnp.float32)
    for _ in range(3):
        g = 0.5 * (g + x / g)
    return g


def _sc_knn(d2m, xqt, ctab):
    mesh = plsc.VectorSubcoreMesh(core_axis_name="c", subcore_axis_name="s")

    @functools.partial(
        pl.kernel,
        mesh=mesh,
        out_type=[
            jax.ShapeDtypeStruct((3, NQ), jnp.float32),
            jax.ShapeDtypeStruct((NQ,), jnp.float32),
        ],
        scratch_types=[
            pltpu.VMEM((CHUNK, QPW), jnp.float32),
            pltpu.VMEM((CHUNK, QPW), jnp.float32),
            pltpu.VMEM((3, QPW), jnp.float32),
            pltpu.VMEM((4 * NGRP * 16,), jnp.float32),
            pltpu.VMEM((4 * NGRP * 16,), jnp.int32),
            pltpu.VMEM((4 * NGRP * 16,), jnp.float32),
            pltpu.VMEM((4, QPW), jnp.int32),
            pltpu.VMEM((512, 128), jnp.float32),
            pltpu.VMEM((3, QPW), jnp.float32),
            pltpu.VMEM((QPW,), jnp.float32),
            pltpu.SemaphoreType.DMA,
            pltpu.SemaphoreType.DMA,
            pltpu.SemaphoreType.DMA,
        ],
    )
    def body(d_hbm, q_hbm, ctab_hbm, oxyz_hbm, oval_hbm,
             dbuf0, dbuf1, qbuf, st_d, st_i, dist_s, idxf, gbuf,
             outx, outv, sem0, sem1, gsem):
        wid = lax.axis_index("s") * 2 + lax.axis_index("c")
        base = wid * QPW
        lanes = lax.iota(jnp.int32, 16)

        pltpu.sync_copy(q_hbm.at[:, pl.ds(base, QPW)], qbuf)
        for gi in range(4 * NGRP):
            st_d[pl.ds(gi * 16, 16)] = jnp.full((16,), BIG, jnp.float32)
            st_i[pl.ds(gi * 16, 16)] = jnp.zeros((16,), jnp.int32)

        def start_chunk(c, buf, sem):
            pltpu.async_copy(
                d_hbm.at[pl.ds(c * CHUNK, CHUNK), pl.ds(base, QPW)], buf, sem)

        def wait_chunk(buf, sem):
            pltpu.make_async_copy(
                d_hbm.at[pl.ds(0, CHUNK), pl.ds(base, QPW)], buf, sem).wait()

        start_chunk(0, dbuf0, sem0)
        start_chunk(1, dbuf1, sem1)

        def process(buf, rowbase, gp):
            NI = 8
            g16s = [gp * 16 * NI + 16 * t for t in range(NI)]
            bases = [gp * 64 * NI + 64 * t for t in range(NI)]
            st = []
            for b in bases:
                st += [st_d[pl.ds(b + 16 * r, 16)] for r in range(4)]
                st += [st_i[pl.ds(b + 16 * r, 16)] for r in range(4)]

            def ins2(j2, carry):
                s = list(carry[:-1])
                jv = carry[-1]
                jb = j2 * 2
                for u in range(2):
                    for t in range(NI):
                        o = t * 8
                        d0, d1, d2, d3, i0, i1, i2, i3 = s[o:o + 8]
                        dv = buf[jb + u, pl.ds(g16s[t], 16)]
                        c0 = dv < d0
                        c1 = dv < d1
                        c2 = dv < d2
                        c3 = dv < d3
                        s[o + 3] = jnp.where(c2, d2, jnp.where(c3, dv, d3))
                        s[o + 7] = jnp.where(c2, i2, jnp.where(c3, jv, i3))
                        s[o + 2] = jnp.where(c1, d1, jnp.where(c2, dv, d2))
                        s[o + 6] = jnp.where(c1, i1, jnp.where(c2, jv, i2))
                        s[o + 1] = jnp.where(c0, d0, jnp.where(c1, dv, d1))
                        s[o + 5] = jnp.where(c0, i0, jnp.where(c1, jv, i1))
                        s[o + 0] = jnp.where(c0, dv, d0)
                        s[o + 4] = jnp.where(c0, jv, i0)
                    jv = jv + 1
                return (*s, jv)

            jv0 = _splat_i32(rowbase)
            out = lax.fori_loop(0, CHUNK // 2, ins2, (*st, jv0))
            for t in range(NI):
                b = bases[t]
                o = t * 8
                for r in range(4):
                    st_d[pl.ds(b + 16 * r, 16)] = out[o + r]
                    st_i[pl.ds(b + 16 * r, 16)] = out[o + 4 + r]

        def chunk_pair(p, _):
            c0 = 2 * p
            wait_chunk(dbuf0, sem0)
            lax.fori_loop(0, NGRP // 8,
                          lambda g, _: (process(dbuf0, c0 * CHUNK, g), 0)[1], 0)

            @pl.when(c0 + 2 < NCH)
            def _():
                start_chunk(c0 + 2, dbuf0, sem0)

            wait_chunk(dbuf1, sem1)
            lax.fori_loop(0, NGRP // 8,
                          lambda g, _: (process(dbuf1, (c0 + 1) * CHUNK, g), 0)[1], 0)

            @pl.when(c0 + 3 < NCH)
            def _():
                start_chunk(c0 + 3, dbuf1, sem1)
            return 0

        lax.fori_loop(0, NCH // 2, chunk_pair, 0)

        def ph2(g, _):
            for k4 in range(4):
                dk = st_d[pl.ds((g * 4 + k4) * 16, 16)]
                dist_s[pl.ds((g * 4 + k4) * 16, 16)] = _newton_sqrt(dk)
                idxf[k4, pl.ds(g * 16, 16)] = st_i[pl.ds((g * 4 + k4) * 16, 16)]
            return 0

        lax.fori_loop(0, NGRP, ph2, 0)

        for half in range(2):
            handles = []
            for k4 in range(4):
                idxslice = idxf.at[k4, pl.ds(half * 128, 128)]
                handles.append(pltpu.async_copy(
                    ctab_hbm.at[idxslice], gbuf.at[pl.ds(k4 * 128, 128)],
                    gsem))
            for h in handles:
                h.wait()

            def ph3(gl, _):
                g = half * 8 + gl
                g16 = g * 16
                l16 = gl * 16
                one = jnp.full((16,), 1.0, jnp.float32)
                zero = jnp.zeros((16,), jnp.float32)

                def sq_body(qi, carry):
                    s1v, s2v, s3v = carry
                    r0 = l16 + qi
                    l0a = gbuf[r0, pl.ds(0, 16)]
                    l0b = gbuf[r0, pl.ds(16, 16)]
                    lsel = lanes == _splat_i32(qi)
                    la = gbuf[128 + r0, pl.ds(0, 16)]
                    lb = gbuf[128 + r0, pl.ds(16, 16)]
                    t1 = _allsum(jnp.abs(la - l0a) + jnp.abs(lb - l0b))
                    la = gbuf[256 + r0, pl.ds(0, 16)]
                    lb = gbuf[256 + r0, pl.ds(16, 16)]
                    t2 = _allsum(jnp.abs(la - l0a) + jnp.abs(lb - l0b))
                    la = gbuf[384 + r0, pl.ds(0, 16)]
                    lb = gbuf[384 + r0, pl.ds(16, 16)]
                    t3 = _allsum(jnp.abs(la - l0a) + jnp.abs(lb - l0b))
                    return (jnp.where(lsel, t1, s1v),
                            jnp.where(lsel, t2, s2v),
                            jnp.where(lsel, t3, s3v))

                s1v, s2v, s3v = lax.fori_loop(0, 16, sq_body,
                                              (zero, zero, zero))
                wst2 = jnp.float32(2.0 * 0.1 ** 2)
                m1 = jnp.where(jnp.exp(-s1v / wst2) > 0.9, one, zero)
                m2 = jnp.where(jnp.exp(-s2v / wst2) > 0.9, one, zero)
                m3 = jnp.where(jnp.exp(-s3v / wst2) > 0.9, one, zero)
                dd0 = dist_s[pl.ds((g * 4 + 0) * 16, 16)]
                dd1 = dist_s[pl.ds((g * 4 + 1) * 16, 16)]
                dd2 = dist_s[pl.ds((g * 4 + 2) * 16, 16)]
                dd3 = dist_s[pl.ds((g * 4 + 3) * 16, 16)]
                w0 = jnp.exp(-dd0)
                w1 = jnp.exp(-dd1) * m1
                w2 = jnp.exp(-dd2) * m2
                w3 = jnp.exp(-dd3) * m3
                ws = (w0 + w1) + (w2 + w3)
                w0 = w0 / ws
                w1 = w1 / ws
                w2 = w2 / ws
                w3 = w3 / ws
                xd = ((w0 * dd0 + w1 * dd1) + w2 * dd2) + w3 * dd3
                outv[pl.ds(g16, 16)] = jnp.where(xd < 0.2, one, zero)
                qr0 = qbuf[0, pl.ds(g16, 16)]
                qr1 = qbuf[1, pl.ds(g16, 16)]
                qr2 = qbuf[2, pl.ds(g16, 16)]
                emod = lanes & 3

                def blend_body(qi, carry):
                    oxx, oxy, oxz = carry
                    r0 = l16 + qi
                    acc = ((_bcast_lane(w0, qi) * gbuf[r0, pl.ds(32, 16)]
                            + _bcast_lane(w1, qi) * gbuf[128 + r0, pl.ds(32, 16)])
                           + _bcast_lane(w2, qi) * gbuf[256 + r0, pl.ds(32, 16)]) \
                        + _bcast_lane(w3, qi) * gbuf[384 + r0, pl.ds(32, 16)]
                    hv = jnp.where(emod == 0, _bcast_lane(qr0, qi),
                                   jnp.where(emod == 1, _bcast_lane(qr1, qi),
                                             jnp.where(emod == 2,
                                                       _bcast_lane(qr2, qi),
                                                       one)))
                    p = acc * hv
                    p = p + _take(p, (lanes + 1) & 15)
                    p = p + _take(p, (lanes + 2) & 15)
                    lsel = lanes == _splat_i32(qi)
                    return (jnp.where(lsel, _bcast_lane(p, 0), oxx),
                            jnp.where(lsel, _bcast_lane(p, 4), oxy),
                            jnp.where(lsel, _bcast_lane(p, 8), oxz))

                oxx, oxy, oxz = lax.fori_loop(0, 16, blend_body,
                                              (zero, zero, zero))
                outx[0, pl.ds(g16, 16)] = oxx
                outx[1, pl.ds(g16, 16)] = oxy
                outx[2, pl.ds(g16, 16)] = oxz
                return 0

            lax.fori_loop(0, 8, ph3, 0)

        pltpu.sync_copy(outx, oxyz_hbm.at[:, pl.ds(base, QPW)])
        pltpu.sync_copy(outv, oval_hbm.at[pl.ds(base, QPW)])

    return body(d2m, xqt, ctab)


def kernel(xyz, verts, verts_transform_inv, lbs_weights):
    xq = xyz[0]
    vp3 = jnp.concatenate(
        [verts[0], jnp.full((NVP - NV, 3), 1.0e6, jnp.float32)], axis=0)
    xqt = xq.T
    lbsp = jnp.pad(lbs_weights, ((0, NVP - NV), (0, 8)))
    tfp = jnp.pad(verts_transform_inv[0].reshape(NV, 16),
                  ((0, NVP - NV), (0, 0)))
    ctab = jnp.concatenate(
        [lbsp, tfp, jnp.zeros((NVP, 80), jnp.float32)], axis=1)
    d2m = _tc_d2(xqt, vp3)
    outx, outv = _sc_knn(d2m, xqt, ctab)
    return outx.T[None], outv[None, :, None]


# speedup vs baseline: 1.3943x; 1.3943x over previous
import functools
import jax
import jax.numpy as jnp
from jax import lax
from jax.experimental import pallas as pl
from jax.experimental.pallas import tpu as pltpu
from jax.experimental.pallas import tpu_sc as plsc

NQ = 8192
NV = 6890
NVP = 6912
NW = 32
QPW = NQ // NW
NGRP = QPW // 16
CHUNK = 96
NCH = NVP // CHUNK
BIG = 3.0e38


def _d2_kernel(xt_ref, vp_ref, o_ref):
    xt = xt_ref[...]
    vp = vp_ref[...]
    xv = jnp.dot(vp.astype(jnp.bfloat16), xt.astype(jnp.bfloat16),
                 preferred_element_type=jnp.float32)
    x2 = (xt[0:1, :] * xt[0:1, :] + xt[1:2, :] * xt[1:2, :]) \
        + xt[2:3, :] * xt[2:3, :]
    v2 = (vp[:, 0:1] * vp[:, 0:1] + vp[:, 1:2] * vp[:, 1:2]) \
        + vp[:, 2:3] * vp[:, 2:3]
    t = x2 + v2
    o_ref[...] = jnp.maximum(t - 2.0 * xv, 1e-12)


def _tc_d2(xqt, vp3):
    BM, BN = 384, 2048
    return pl.pallas_call(
        _d2_kernel,
        out_shape=jax.ShapeDtypeStruct((NVP, NQ), jnp.float32),
        grid=(NVP // BM, NQ // BN),
        in_specs=[
            pl.BlockSpec((3, BN), lambda i, j: (0, j)),
            pl.BlockSpec((BM, 3), lambda i, j: (i, 0)),
        ],
        out_specs=pl.BlockSpec((BM, BN), lambda i, j: (i, j)),
    )(xqt, vp3)


def _splat_i32(x):
    return jnp.zeros((16,), jnp.int32) + x


def _take(v, idx):
    return lax.gather(
        v, idx[:, None],
        lax.GatherDimensionNumbers(offset_dims=(), collapsed_slice_dims=(0,),
                                   start_index_map=(0,)),
        slice_sizes=(1,), mode=lax.GatherScatterMode.PROMISE_IN_BOUNDS)


def _bcast_lane(v, q):
    return _take(v, _splat_i32(q))


def _allsum(v):
    lanes = lax.iota(jnp.int32, 16)
    for s in (8, 4, 2, 1):
        v = v + _take(v, (lanes + s) & 15)
    return v


def _newton_sqrt(x):
    i = lax.bitcast_convert_type(x, jnp.int32)
    g = lax.bitcast_convert_type((i >> 1) + jnp.int32(0x1FBD1DF5), jnp.float32)
    for _ in range(3):
        g = 0.5 * (g + x / g)
    return g


def _sc_knn(d2m, xqt, ctab):
    mesh = plsc.VectorSubcoreMesh(core_axis_name="c", subcore_axis_name="s")

    @functools.partial(
        pl.kernel,
        mesh=mesh,
        out_type=[
            jax.ShapeDtypeStruct((3, NQ), jnp.float32),
            jax.ShapeDtypeStruct((NQ,), jnp.float32),
        ],
        scratch_types=[
            pltpu.VMEM((CHUNK, QPW), jnp.float32),
            pltpu.VMEM((CHUNK, QPW), jnp.float32),
            pltpu.VMEM((3, QPW), jnp.float32),
            pltpu.VMEM((4 * NGRP * 16,), jnp.float32),
            pltpu.VMEM((4 * NGRP * 16,), jnp.int32),
            pltpu.VMEM((4 * NGRP * 16,), jnp.float32),
            pltpu.VMEM((4, QPW), jnp.int32),
            pltpu.VMEM((512, 128), jnp.float32),
            pltpu.VMEM((3, QPW), jnp.float32),
            pltpu.VMEM((QPW,), jnp.float32),
            pltpu.SemaphoreType.DMA,
            pltpu.SemaphoreType.DMA,
            pltpu.SemaphoreType.DMA,
        ],
    )
    def body(d_hbm, q_hbm, ctab_hbm, oxyz_hbm, oval_hbm,
             dbuf0, dbuf1, qbuf, st_d, st_i, dist_s, idxf, gbuf,
             outx, outv, sem0, sem1, gsem):
        wid = lax.axis_index("s") * 2 + lax.axis_index("c")
        base = wid * QPW
        lanes = lax.iota(jnp.int32, 16)

        pltpu.sync_copy(q_hbm.at[:, pl.ds(base, QPW)], qbuf)
        for gi in range(4 * NGRP):
            st_d[pl.ds(gi * 16, 16)] = jnp.full((16,), BIG, jnp.float32)
            st_i[pl.ds(gi * 16, 16)] = jnp.zeros((16,), jnp.int32)

        def start_chunk(c, buf, sem):
            pltpu.async_copy(
                d_hbm.at[pl.ds(c * CHUNK, CHUNK), pl.ds(base, QPW)], buf, sem)

        def wait_chunk(buf, sem):
            pltpu.make_async_copy(
                d_hbm.at[pl.ds(0, CHUNK), pl.ds(base, QPW)], buf, sem).wait()

        start_chunk(0, dbuf0, sem0)
        start_chunk(1, dbuf1, sem1)

        def process(buf, rowbase, gp):
            NI = 4
            g16s = [gp * 16 * NI + 16 * t for t in range(NI)]
            bases = [gp * 64 * NI + 64 * t for t in range(NI)]
            st = []
            for b in bases:
                st += [st_d[pl.ds(b + 16 * r, 16)] for r in range(4)]
                st += [st_i[pl.ds(b + 16 * r, 16)] for r in range(4)]

            def ins2(j2, carry):
                s = list(carry[:-1])
                jv = carry[-1]
                jb = j2 * 2
                for u in range(2):
                    for t in range(NI):
                        o = t * 8
                        d0, d1, d2, d3, i0, i1, i2, i3 = s[o:o + 8]
                        dv = buf[jb + u, pl.ds(g16s[t], 16)]
                        c0 = dv < d0
                        c1 = dv < d1
                        c2 = dv < d2
                        c3 = dv < d3
                        s[o + 3] = jnp.where(c2, d2, jnp.where(c3, dv, d3))
                        s[o + 7] = jnp.where(c2, i2, jnp.where(c3, jv, i3))
                        s[o + 2] = jnp.where(c1, d1, jnp.where(c2, dv, d2))
                        s[o + 6] = jnp.where(c1, i1, jnp.where(c2, jv, i2))
                        s[o + 1] = jnp.where(c0, d0, jnp.where(c1, dv, d1))
                        s[o + 5] = jnp.where(c0, i0, jnp.where(c1, jv, i1))
                        s[o + 0] = jnp.where(c0, dv, d0)
                        s[o + 4] = jnp.where(c0, jv, i0)
                    jv = jv + 1
                return (*s, jv)

            jv0 = _splat_i32(rowbase)
            out = lax.fori_loop(0, CHUNK // 2, ins2, (*st, jv0))
            for t in range(NI):
                b = bases[t]
                o = t * 8
                for r in range(4):
                    st_d[pl.ds(b + 16 * r, 16)] = out[o + r]
                    st_i[pl.ds(b + 16 * r, 16)] = out[o + 4 + r]

        def chunk_pair(p, _):
            c0 = 2 * p
            wait_chunk(dbuf0, sem0)
            lax.fori_loop(0, NGRP // 4,
                          lambda g, _: (process(dbuf0, c0 * CHUNK, g), 0)[1], 0)

            @pl.when(c0 + 2 < NCH)
            def _():
                start_chunk(c0 + 2, dbuf0, sem0)

            wait_chunk(dbuf1, sem1)
            lax.fori_loop(0, NGRP // 4,
                          lambda g, _: (process(dbuf1, (c0 + 1) * CHUNK, g), 0)[1], 0)

            @pl.when(c0 + 3 < NCH)
            def _():
                start_chunk(c0 + 3, dbuf1, sem1)
            return 0

        lax.fori_loop(0, NCH // 2, chunk_pair, 0)

        def ph2(g, _):
            for k4 in range(4):
                dk = st_d[pl.ds((g * 4 + k4) * 16, 16)]
                dist_s[pl.ds((g * 4 + k4) * 16, 16)] = _newton_sqrt(dk)
                idxf[k4, pl.ds(g * 16, 16)] = st_i[pl.ds((g * 4 + k4) * 16, 16)]
            return 0

        lax.fori_loop(0, NGRP, ph2, 0)

        for half in range(2):
            handles = []
            for k4 in range(4):
                idxslice = idxf.at[k4, pl.ds(half * 128, 128)]
                handles.append(pltpu.async_copy(
                    ctab_hbm.at[idxslice], gbuf.at[pl.ds(k4 * 128, 128)],
                    gsem))
            for h in handles:
                h.wait()

            def ph3(gl, _):
                g = half * 8 + gl
                g16 = g * 16
                l16 = gl * 16
                one = jnp.full((16,), 1.0, jnp.float32)
                zero = jnp.zeros((16,), jnp.float32)

                def sq_body(qi, carry):
                    s1v, s2v, s3v = carry
                    r0 = l16 + qi
                    l0a = gbuf[r0, pl.ds(0, 16)]
                    l0b = gbuf[r0, pl.ds(16, 16)]
                    lsel = lanes == _splat_i32(qi)
                    la = gbuf[128 + r0, pl.ds(0, 16)]
                    lb = gbuf[128 + r0, pl.ds(16, 16)]
                    t1 = _allsum(jnp.abs(la - l0a) + jnp.abs(lb - l0b))
                    la = gbuf[256 + r0, pl.ds(0, 16)]
                    lb = gbuf[256 + r0, pl.ds(16, 16)]
                    t2 = _allsum(jnp.abs(la - l0a) + jnp.abs(lb - l0b))
                    la = gbuf[384 + r0, pl.ds(0, 16)]
                    lb = gbuf[384 + r0, pl.ds(16, 16)]
                    t3 = _allsum(jnp.abs(la - l0a) + jnp.abs(lb - l0b))
                    return (jnp.where(lsel, t1, s1v),
                            jnp.where(lsel, t2, s2v),
                            jnp.where(lsel, t3, s3v))

                s1v, s2v, s3v = lax.fori_loop(0, 16, sq_body,
                                              (zero, zero, zero))
                wst2 = jnp.float32(2.0 * 0.1 ** 2)
                m1 = jnp.where(jnp.exp(-s1v / wst2) > 0.9, one, zero)
                m2 = jnp.where(jnp.exp(-s2v / wst2) > 0.9, one, zero)
                m3 = jnp.where(jnp.exp(-s3v / wst2) > 0.9, one, zero)
                dd0 = dist_s[pl.ds((g * 4 + 0) * 16, 16)]
                dd1 = dist_s[pl.ds((g * 4 + 1) * 16, 16)]
                dd2 = dist_s[pl.ds((g * 4 + 2) * 16, 16)]
                dd3 = dist_s[pl.ds((g * 4 + 3) * 16, 16)]
                w0 = jnp.exp(-dd0)
                w1 = jnp.exp(-dd1) * m1
                w2 = jnp.exp(-dd2) * m2
                w3 = jnp.exp(-dd3) * m3
                ws = (w0 + w1) + (w2 + w3)
                w0 = w0 / ws
                w1 = w1 / ws
                w2 = w2 / ws
                w3 = w3 / ws
                xd = ((w0 * dd0 + w1 * dd1) + w2 * dd2) + w3 * dd3
                outv[pl.ds(g16, 16)] = jnp.where(xd < 0.2, one, zero)
                qr0 = qbuf[0, pl.ds(g16, 16)]
                qr1 = qbuf[1, pl.ds(g16, 16)]
                qr2 = qbuf[2, pl.ds(g16, 16)]
                emod = lanes & 3

                def blend_body(qi, carry):
                    oxx, oxy, oxz = carry
                    r0 = l16 + qi
                    acc = ((_bcast_lane(w0, qi) * gbuf[r0, pl.ds(32, 16)]
                            + _bcast_lane(w1, qi) * gbuf[128 + r0, pl.ds(32, 16)])
                           + _bcast_lane(w2, qi) * gbuf[256 + r0, pl.ds(32, 16)]) \
                        + _bcast_lane(w3, qi) * gbuf[384 + r0, pl.ds(32, 16)]
                    hv = jnp.where(emod == 0, _bcast_lane(qr0, qi),
                                   jnp.where(emod == 1, _bcast_lane(qr1, qi),
                                             jnp.where(emod == 2,
                                                       _bcast_lane(qr2, qi),
                                                       one)))
                    p = acc * hv
                    p = p + _take(p, (lanes + 1) & 15)
                    p = p + _take(p, (lanes + 2) & 15)
                    lsel = lanes == _splat_i32(qi)
                    return (jnp.where(lsel, _bcast_lane(p, 0), oxx),
                            jnp.where(lsel, _bcast_lane(p, 4), oxy),
                            jnp.where(lsel, _bcast_lane(p, 8), oxz))

                oxx, oxy, oxz = lax.fori_loop(0, 16, blend_body,
                                              (zero, zero, zero))
                outx[0, pl.ds(g16, 16)] = oxx
                outx[1, pl.ds(g16, 16)] = oxy
                outx[2, pl.ds(g16, 16)] = oxz
                return 0

            lax.fori_loop(0, 8, ph3, 0)

        pltpu.sync_copy(outx, oxyz_hbm.at[:, pl.ds(base, QPW)])
        pltpu.sync_copy(outv, oval_hbm.at[pl.ds(base, QPW)])

    return body(d2m, xqt, ctab)


def kernel(xyz, verts, verts_transform_inv, lbs_weights):
    xq = xyz[0]
    vp3 = jnp.concatenate(
        [verts[0], jnp.full((NVP - NV, 3), 1.0e6, jnp.float32)], axis=0)
    xqt = xq.T
    lbsp = jnp.pad(lbs_weights, ((0, NVP - NV), (0, 8)))
    tfp = jnp.pad(verts_transform_inv[0].reshape(NV, 16),
                  ((0, NVP - NV), (0, 0)))
    ctab = jnp.concatenate(
        [lbsp, tfp, jnp.zeros((NVP, 80), jnp.float32)], axis=1)
    d2m = _tc_d2(xqt, vp3)
    outx, outv = _sc_knn(d2m, xqt, ctab)
    return outx.T[None], outv[None, :, None]
